# baseline (speedup 1.0000x reference)
import functools
import jax
import jax.numpy as jnp
from jax import lax
from jax.experimental import pallas as pl
from jax.experimental.pallas import tpu as pltpu
from jax.experimental.pallas import tpu_sc as plsc

_NUM_ACTIONS = 1000000
_LATENT_DIM = 16
_BATCH = 16384

_NC = 2
_NS = 16
_NW = _NC * _NS
_B_PER_W = _BATCH // _NW
_L = 16

_mesh = plsc.VectorSubcoreMesh(core_axis_name="c", subcore_axis_name="s")


@functools.partial(
    pl.kernel,
    mesh=_mesh,
    out_type=jax.ShapeDtypeStruct((_LATENT_DIM, _BATCH), jnp.float32),
    scratch_types=[
        pltpu.VMEM((_B_PER_W,), jnp.int32),
        pltpu.VMEM((_LATENT_DIM * _B_PER_W,), jnp.int32),
        pltpu.VMEM((_LATENT_DIM, _B_PER_W), jnp.float32),
        pltpu.SemaphoreType.DMA,
    ],
    compiler_params=pltpu.CompilerParams(use_tc_tiling_on_sc=False),
)
def _gather_kernel(idx_hbm, table_flat_hbm, out_t_hbm, idx_v, offs_v, block_v,
                   sem):
    wid = lax.axis_index("s") * _NC + lax.axis_index("c")
    base = wid * _B_PER_W
    pltpu.sync_copy(idx_hbm.at[pl.ds(base, _B_PER_W)], idx_v)

    n_chunks = _B_PER_W // _L

    @pl.loop(0, n_chunks)
    def _compute_offsets(k):
        r = idx_v[pl.ds(k * _L, _L)]
        for c in range(_LATENT_DIM):
            offs_v[pl.ds(c * _B_PER_W + k * _L, _L)] = r + c * _NUM_ACTIONS

    for c in range(_LATENT_DIM):
        pltpu.async_copy(
            table_flat_hbm.at[offs_v.at[pl.ds(c * _B_PER_W, _B_PER_W)]],
            block_v.at[c],
            sem,
        )
    for c in range(_LATENT_DIM):
        pltpu.make_async_copy(
            table_flat_hbm.at[offs_v.at[pl.ds(c * _B_PER_W, _B_PER_W)]],
            block_v.at[c],
            sem,
        ).wait()

    pltpu.sync_copy(block_v, out_t_hbm.at[:, pl.ds(base, _B_PER_W)])


def kernel(input, action_embedding):
    idx = input.reshape(_BATCH)
    out_t = _gather_kernel(idx, action_embedding.T.reshape(_LATENT_DIM * _NUM_ACTIONS))
    return out_t.T[None, :, :]


# speedup vs baseline: 2.7395x; 2.7395x over previous
import functools
import jax
import jax.numpy as jnp
from jax import lax
from jax.experimental import pallas as pl
from jax.experimental.pallas import tpu as pltpu
from jax.experimental.pallas import tpu_sc as plsc

_NUM_ACTIONS = 1000000
_LATENT_DIM = 16
_BATCH = 16384

_NC = 2
_NS = 16
_NW = _NC * _NS
_B_PER_W = _BATCH // _NW
_L = 16

_mesh = plsc.VectorSubcoreMesh(core_axis_name="c", subcore_axis_name="s")


@functools.partial(
    pl.kernel,
    mesh=_mesh,
    out_type=jax.ShapeDtypeStruct((_LATENT_DIM, _BATCH), jnp.float32),
    scratch_types=[
        pltpu.VMEM((_B_PER_W,), jnp.int32),
        pltpu.VMEM((_LATENT_DIM * _B_PER_W,), jnp.int32),
        pltpu.VMEM((_LATENT_DIM, _B_PER_W), jnp.float32),
        pltpu.SemaphoreType.DMA,
    ],
    compiler_params=pltpu.CompilerParams(use_tc_tiling_on_sc=False),
)
def _gather_kernel(idx_hbm, table_flat_hbm, out_t_hbm, idx_v, offs_v, block_v,
                   sem):
    wid = lax.axis_index("s") * _NC + lax.axis_index("c")
    base = wid * _B_PER_W
    pltpu.sync_copy(idx_hbm.at[pl.ds(base, _B_PER_W)], idx_v)

    n_chunks = _B_PER_W // _L

    @pl.loop(0, n_chunks)
    def _compute_offsets(k):
        r = idx_v[pl.ds(k * _L, _L)]
        for c in range(_LATENT_DIM):
            offs_v[pl.ds(c * _B_PER_W + k * _L, _L)] = r * _LATENT_DIM + c

    for c in range(_LATENT_DIM):
        pltpu.async_copy(
            table_flat_hbm.at[offs_v.at[pl.ds(c * _B_PER_W, _B_PER_W)]],
            block_v.at[c],
            sem,
        )
    for c in range(_LATENT_DIM):
        pltpu.make_async_copy(
            table_flat_hbm.at[offs_v.at[pl.ds(c * _B_PER_W, _B_PER_W)]],
            block_v.at[c],
            sem,
        ).wait()

    pltpu.sync_copy(block_v, out_t_hbm.at[:, pl.ds(base, _B_PER_W)])


def kernel(input, action_embedding):
    idx = input.reshape(_BATCH)
    out_t = _gather_kernel(idx, action_embedding.reshape(_LATENT_DIM * _NUM_ACTIONS))
    return out_t.T[None, :, :]
